# baseline (speedup 1.0000x reference)
import functools

import jax
import jax.numpy as jnp
from jax import lax
from jax.experimental import pallas as pl
from jax.experimental.pallas import tpu as pltpu
from jax.experimental.pallas import tpu_sc as plsc

_NC = 2
_NS = 16
_L = 16
_NW = _NC * _NS

_B = 16384
_D = 64
_PD = 128
_BPW = _B // _NW
_CH = 64
_NCH = _BPW // _CH
_G = _CH // _L


def _neg_sqrt(x):
    i = lax.bitcast_convert_type(x, jnp.int32)
    i = jnp.int32(0x5F3759DF) - lax.shift_right_logical(i, 1)
    y = lax.bitcast_convert_type(i, jnp.float32)
    for _ in range(3):
        y = y * (1.5 - 0.5 * x * y * y)
    return -(x * y)


def _body(uh_h, ph_h, nh_h, rh_h, users_h, pos_h, neg_h, rel_h,
          ent_h, relemb_h, outp_h, outn_h,
          uhs, phs, nhs, rhs, uidx, pidx, nidx, ridx,
          bufs, op_v, on_v, sem0, sem1):
    wid = lax.axis_index("s") * _NC + lax.axis_index("c")
    base = wid * _BPW
    pltpu.sync_copy(uh_h.at[pl.ds(base, _BPW)], uhs)
    pltpu.sync_copy(ph_h.at[pl.ds(base, _BPW)], phs)
    pltpu.sync_copy(nh_h.at[pl.ds(base, _BPW)], nhs)
    pltpu.sync_copy(rh_h.at[pl.ds(base, _BPW)], rhs)
    pltpu.sync_copy(users_h.at[pl.ds(base, _BPW)], uidx)
    pltpu.sync_copy(pos_h.at[pl.ds(base, _BPW)], pidx)
    pltpu.sync_copy(neg_h.at[pl.ds(base, _BPW)], nidx)
    pltpu.sync_copy(rel_h.at[pl.ds(base, _BPW)], ridx)

    sems = (sem0, sem1)
    lane = lax.iota(jnp.int32, _L)
    zero = jnp.zeros((_L,), jnp.float32)
    one = jnp.ones((_L,), jnp.int32)

    def fire(c, slot):
        off = c * _CH
        sem = sems[slot]
        u_b, r_b, p_b, n_b = (bufs.at[4 * slot + j] for j in range(4))
        cp0 = pltpu.async_copy(ent_h.at[uhs.at[pl.ds(off, _CH)]], u_b, sem)
        cp1 = pltpu.async_copy(relemb_h.at[rhs.at[pl.ds(off, _CH)]], r_b, sem)
        cp2 = pltpu.async_copy(ent_h.at[phs.at[pl.ds(off, _CH)]], p_b, sem)
        cp3 = pltpu.async_copy(ent_h.at[nhs.at[pl.ds(off, _CH)]], n_b, sem)
        return cp0, cp1, cp2, cp3

    pend = fire(0, 0)
    for c in range(_NCH):
        slot = c % 2
        nxt = fire(c + 1, 1 - slot) if c + 1 < _NCH else None
        for cp in pend:
            cp.wait()
        pend = nxt
        off = c * _CH
        u_b, r_b, p_b, n_b = (bufs.at[4 * slot + j] for j in range(4))

        def group(g, _):
            row = g * _L + lane
            sl = pl.ds(off + g * _L, _L)
            cbu = (uidx[sl] & one) * _D
            cbr = (ridx[sl] & one) * _D
            cbp = (pidx[sl] & one) * _D
            cbn = (nidx[sl] & one) * _D

            def dblock(db, carry):
                ap, an = carry
                d0 = db * _L
                for k in range(_L):
                    col = jnp.full((_L,), d0 + k, jnp.int32)
                    uv = plsc.load_gather(u_b, [row, cbu + col])
                    rv = plsc.load_gather(r_b, [row, cbr + col])
                    pv = plsc.load_gather(p_b, [row, cbp + col])
                    nv = plsc.load_gather(n_b, [row, cbn + col])
                    t = uv + rv
                    dp = t - pv
                    dn = t - nv
                    ap = ap + dp * dp
                    an = an + dn * dn
                return ap, an

            ap, an = lax.fori_loop(0, _D // _L, dblock, (zero, zero))
            op_v[sl] = _neg_sqrt(ap)
            on_v[sl] = _neg_sqrt(an)
            return 0

        lax.fori_loop(0, _G, group, 0)

    pltpu.sync_copy(op_v, outp_h.at[pl.ds(base, _BPW)])
    pltpu.sync_copy(on_v, outn_h.at[pl.ds(base, _BPW)])


_sc_score = functools.partial(
    pl.kernel,
    out_type=(jax.ShapeDtypeStruct((_B,), jnp.float32),
              jax.ShapeDtypeStruct((_B,), jnp.float32)),
    mesh=plsc.VectorSubcoreMesh(core_axis_name="c", subcore_axis_name="s"),
    compiler_params=pltpu.CompilerParams(needs_layout_passes=False),
    scratch_types=[
        pltpu.VMEM((_BPW,), jnp.int32),
        pltpu.VMEM((_BPW,), jnp.int32),
        pltpu.VMEM((_BPW,), jnp.int32),
        pltpu.VMEM((_BPW,), jnp.int32),
        pltpu.VMEM((_BPW,), jnp.int32),
        pltpu.VMEM((_BPW,), jnp.int32),
        pltpu.VMEM((_BPW,), jnp.int32),
        pltpu.VMEM((_BPW,), jnp.int32),
        pltpu.VMEM((8, _CH, _PD), jnp.float32),
        pltpu.VMEM((_BPW,), jnp.float32),
        pltpu.VMEM((_BPW,), jnp.float32),
        pltpu.SemaphoreType.DMA,
        pltpu.SemaphoreType.DMA,
    ],
)(_body)


def kernel(users, pos_items, neg_items, relations, ent_emb, rel_emb):
    users = users.astype(jnp.int32)
    pos_items = pos_items.astype(jnp.int32)
    neg_items = neg_items.astype(jnp.int32)
    relations = relations.astype(jnp.int32)
    ent2 = ent_emb.reshape(ent_emb.shape[0] // 2, 2 * _D)
    rel2 = rel_emb.reshape(rel_emb.shape[0] // 2, 2 * _D)
    return _sc_score(
        lax.shift_right_logical(users, 1), lax.shift_right_logical(pos_items, 1),
        lax.shift_right_logical(neg_items, 1), lax.shift_right_logical(relations, 1),
        users, pos_items, neg_items, relations, ent2, rel2)


# speedup vs baseline: 2.1763x; 2.1763x over previous
import functools

import jax
import jax.numpy as jnp
from jax import lax
from jax.experimental import pallas as pl
from jax.experimental.pallas import tpu as pltpu
from jax.experimental.pallas import tpu_sc as plsc

_NC = 2
_NS = 16
_L = 16
_NW = _NC * _NS

_B = 16384
_D = 64
_NR = 1000
_BPW = _B // _NW
_CH = 64
_NCH = _BPW // _CH
_G = _CH // _L


def _neg_sqrt(x):
    i = lax.bitcast_convert_type(x, jnp.int32)
    i = jnp.int32(0x5F3759DF) - lax.shift_right_logical(i, 1)
    y = lax.bitcast_convert_type(i, jnp.float32)
    for _ in range(3):
        y = y * (1.5 - 0.5 * x * y * y)
    return -(x * y)


def _body(users_h, pos_h, neg_h, rel_h, ent_h, relemb_h, outp_h, outn_h,
          uidx, pidx, nidx, ridx, bufs, op_v, on_v, sem0, sem1):
    wid = lax.axis_index("s") * _NC + lax.axis_index("c")
    base = wid * _BPW
    pltpu.sync_copy(users_h.at[pl.ds(base, _BPW)], uidx)
    pltpu.sync_copy(pos_h.at[pl.ds(base, _BPW)], pidx)
    pltpu.sync_copy(neg_h.at[pl.ds(base, _BPW)], nidx)
    pltpu.sync_copy(rel_h.at[pl.ds(base, _BPW)], ridx)

    sems = (sem0, sem1)
    lane = lax.iota(jnp.int32, _L)
    zero = jnp.zeros((_L,), jnp.float32)

    def fire(c, slot):
        off = c * _CH
        sem = sems[slot]
        u_b, r_b, p_b, n_b = (bufs.at[4 * slot + j] for j in range(4))

        def grp(g, _):
            s = pl.ds(off + g * _L, _L)
            uv = uidx[s]
            rv = ridx[s]
            pv = pidx[s]
            nv = nidx[s]
            for k in range(_L):
                d = pl.ds(g * _L + k, 1)
                ue, re_, pe, ne = uv[k], rv[k], pv[k], nv[k]
                pltpu.async_copy(
                    ent_h.at[lax.shift_right_logical(ue, 3)].at[pl.ds(ue & 7, 1)],
                    u_b.at[d], sem)
                pltpu.async_copy(
                    relemb_h.at[lax.shift_right_logical(re_, 3)].at[pl.ds(re_ & 7, 1)],
                    r_b.at[d], sem)
                pltpu.async_copy(
                    ent_h.at[lax.shift_right_logical(pe, 3)].at[pl.ds(pe & 7, 1)],
                    p_b.at[d], sem)
                pltpu.async_copy(
                    ent_h.at[lax.shift_right_logical(ne, 3)].at[pl.ds(ne & 7, 1)],
                    n_b.at[d], sem)
            return 0

        lax.fori_loop(0, _G, grp, 0)

    def drain(slot):
        sem = sems[slot]
        for j in range(4):
            pltpu.make_async_copy(ent_h.at[pl.ds(0, _CH // 8)],
                                  bufs.at[4 * slot + j].reshape(_CH // 8, 8, _D),
                                  sem).wait()

    fire(0, 0)
    for c in range(_NCH):
        slot = c % 2
        if c + 1 < _NCH:
            fire(c + 1, 1 - slot)
        drain(slot)
        off = c * _CH
        u_b, r_b, p_b, n_b = (bufs.at[4 * slot + j] for j in range(4))

        def group(g, _):
            row = g * _L + lane
            sl = pl.ds(off + g * _L, _L)

            def dblock(db, carry):
                ap, an = carry
                d0 = db * _L
                for k in range(_L):
                    col = jnp.full((_L,), d0 + k, jnp.int32)
                    uv = plsc.load_gather(u_b, [row, col])
                    rv = plsc.load_gather(r_b, [row, col])
                    pv = plsc.load_gather(p_b, [row, col])
                    nv = plsc.load_gather(n_b, [row, col])
                    t = uv + rv
                    dp = t - pv
                    dn = t - nv
                    ap = ap + dp * dp
                    an = an + dn * dn
                return ap, an

            ap, an = lax.fori_loop(0, _D // _L, dblock, (zero, zero))
            op_v[sl] = _neg_sqrt(ap)
            on_v[sl] = _neg_sqrt(an)
            return 0

        lax.fori_loop(0, _G, group, 0)

    pltpu.sync_copy(op_v, outp_h.at[pl.ds(base, _BPW)])
    pltpu.sync_copy(on_v, outn_h.at[pl.ds(base, _BPW)])


_sc_score = functools.partial(
    pl.kernel,
    out_type=(jax.ShapeDtypeStruct((_B,), jnp.float32),
              jax.ShapeDtypeStruct((_B,), jnp.float32)),
    mesh=plsc.VectorSubcoreMesh(core_axis_name="c", subcore_axis_name="s"),
    compiler_params=pltpu.CompilerParams(needs_layout_passes=False,
                                         disable_bounds_checks=True),
    scratch_types=[
        pltpu.VMEM((_BPW,), jnp.int32),
        pltpu.VMEM((_BPW,), jnp.int32),
        pltpu.VMEM((_BPW,), jnp.int32),
        pltpu.VMEM((_BPW,), jnp.int32),
        pltpu.VMEM((8, _CH, _D), jnp.float32),
        pltpu.VMEM((_BPW,), jnp.float32),
        pltpu.VMEM((_BPW,), jnp.float32),
        pltpu.SemaphoreType.DMA,
        pltpu.SemaphoreType.DMA,
    ],
)(_body)


def kernel(users, pos_items, neg_items, relations, ent_emb, rel_emb):
    ent3 = ent_emb.reshape(ent_emb.shape[0] // 8, 8, _D)
    rel3 = rel_emb.reshape(rel_emb.shape[0] // 8, 8, _D)
    return _sc_score(users.astype(jnp.int32), pos_items.astype(jnp.int32),
                     neg_items.astype(jnp.int32), relations.astype(jnp.int32),
                     ent3, rel3)


# speedup vs baseline: 2.6090x; 1.1988x over previous
import functools

import jax
import jax.numpy as jnp
from jax import lax
from jax.experimental import pallas as pl
from jax.experimental.pallas import tpu as pltpu
from jax.experimental.pallas import tpu_sc as plsc

_NC = 2
_NS = 16
_L = 16
_NW = _NC * _NS

_B = 16384
_D = 64
_NR = 1000
_BPW = _B // _NW
_CH = 64
_NCH = _BPW // _CH
_G = _CH // _L


def _neg_sqrt(x):
    i = lax.bitcast_convert_type(x, jnp.int32)
    i = jnp.int32(0x5F3759DF) - lax.shift_right_logical(i, 1)
    y = lax.bitcast_convert_type(i, jnp.float32)
    for _ in range(3):
        y = y * (1.5 - 0.5 * x * y * y)
    return -(x * y)


def _body(users_h, pos_h, neg_h, rel_h, ent_h, relemb_h, outp_h, outn_h,
          uidx, pidx, nidx, ridx, bufs, op_v, on_v, sem0, sem1):
    wid = lax.axis_index("s") * _NC + lax.axis_index("c")
    base = wid * _BPW
    pltpu.sync_copy(users_h.at[pl.ds(base, _BPW)], uidx)
    pltpu.sync_copy(pos_h.at[pl.ds(base, _BPW)], pidx)
    pltpu.sync_copy(neg_h.at[pl.ds(base, _BPW)], nidx)
    pltpu.sync_copy(rel_h.at[pl.ds(base, _BPW)], ridx)

    sems = (sem0, sem1)
    lane = lax.iota(jnp.int32, _L)
    zero = jnp.zeros((_L,), jnp.float32)

    def fire(c, slot):
        off = c * _CH
        sem = sems[slot]
        u_b, r_b, p_b, n_b = (bufs.at[4 * slot + j] for j in range(4))

        def grp(g, _):
            s = pl.ds(off + g * _L, _L)
            uv = uidx[s]
            rv = ridx[s]
            pv = pidx[s]
            nv = nidx[s]
            for k in range(_L):
                d = pl.ds(g * _L + k, 1)
                ue, re_, pe, ne = uv[k], rv[k], pv[k], nv[k]
                pltpu.async_copy(
                    ent_h.at[lax.shift_right_logical(ue, 3)].at[pl.ds(ue & 7, 1)],
                    u_b.at[d], sem)
                pltpu.async_copy(
                    relemb_h.at[lax.shift_right_logical(re_, 3)].at[pl.ds(re_ & 7, 1)],
                    r_b.at[d], sem)
                pltpu.async_copy(
                    ent_h.at[lax.shift_right_logical(pe, 3)].at[pl.ds(pe & 7, 1)],
                    p_b.at[d], sem)
                pltpu.async_copy(
                    ent_h.at[lax.shift_right_logical(ne, 3)].at[pl.ds(ne & 7, 1)],
                    n_b.at[d], sem)
            return 0

        lax.fori_loop(0, _G, grp, 0)

    def drain(slot):
        sem = sems[slot]
        for j in range(4):
            pltpu.make_async_copy(ent_h.at[pl.ds(0, _CH // 8)],
                                  bufs.at[4 * slot + j].reshape(_CH // 8, 8, _D),
                                  sem).wait()

    fire(0, 0)
    for c in range(_NCH):
        slot = c % 2
        if c + 1 < _NCH:
            fire(c + 1, 1 - slot)
        drain(slot)
        off = c * _CH
        u_b, r_b, p_b, n_b = (bufs.at[4 * slot + j] for j in range(4))

        def group(g, _):
            sl = pl.ds(off + g * _L, _L)
            vp = zero
            vn = zero
            for k in range(_L):
                row = g * _L + k
                app = zero
                ann = zero
                for q in range(_D // _L):
                    cs = pl.ds(q * _L, _L)
                    t = u_b[row, cs] + r_b[row, cs]
                    dp = t - p_b[row, cs]
                    dn = t - n_b[row, cs]
                    app = app + dp * dp
                    ann = ann + dn * dn
                sp = jnp.sum(app)
                sn = jnp.sum(ann)
                vp = jnp.where(lane == k, sp, vp)
                vn = jnp.where(lane == k, sn, vn)
            op_v[sl] = _neg_sqrt(vp)
            on_v[sl] = _neg_sqrt(vn)
            return 0

        lax.fori_loop(0, _G, group, 0)

    pltpu.sync_copy(op_v, outp_h.at[pl.ds(base, _BPW)])
    pltpu.sync_copy(on_v, outn_h.at[pl.ds(base, _BPW)])


_sc_score = functools.partial(
    pl.kernel,
    out_type=(jax.ShapeDtypeStruct((_B,), jnp.float32),
              jax.ShapeDtypeStruct((_B,), jnp.float32)),
    mesh=plsc.VectorSubcoreMesh(core_axis_name="c", subcore_axis_name="s"),
    compiler_params=pltpu.CompilerParams(needs_layout_passes=False,
                                         disable_bounds_checks=True),
    scratch_types=[
        pltpu.VMEM((_BPW,), jnp.int32),
        pltpu.VMEM((_BPW,), jnp.int32),
        pltpu.VMEM((_BPW,), jnp.int32),
        pltpu.VMEM((_BPW,), jnp.int32),
        pltpu.VMEM((8, _CH, _D), jnp.float32),
        pltpu.VMEM((_BPW,), jnp.float32),
        pltpu.VMEM((_BPW,), jnp.float32),
        pltpu.SemaphoreType.DMA,
        pltpu.SemaphoreType.DMA,
    ],
)(_body)


def kernel(users, pos_items, neg_items, relations, ent_emb, rel_emb):
    ent3 = ent_emb.reshape(ent_emb.shape[0] // 8, 8, _D)
    rel3 = rel_emb.reshape(rel_emb.shape[0] // 8, 8, _D)
    return _sc_score(users.astype(jnp.int32), pos_items.astype(jnp.int32),
                     neg_items.astype(jnp.int32), relations.astype(jnp.int32),
                     ent3, rel3)


# speedup vs baseline: 2.6168x; 1.0030x over previous
import functools

import jax
import jax.numpy as jnp
from jax import lax
from jax.experimental import pallas as pl
from jax.experimental.pallas import tpu as pltpu
from jax.experimental.pallas import tpu_sc as plsc

_NC = 2
_NS = 16
_L = 16
_NW = _NC * _NS

_B = 16384
_D = 64
_NR = 1000
_BPW = _B // _NW
_CH = 64
_NCH = _BPW // _CH
_G = _CH // _L


def _neg_sqrt(x):
    i = lax.bitcast_convert_type(x, jnp.int32)
    i = jnp.int32(0x5F3759DF) - lax.shift_right_logical(i, 1)
    y = lax.bitcast_convert_type(i, jnp.float32)
    for _ in range(3):
        y = y * (1.5 - 0.5 * x * y * y)
    return -(x * y)


def _body(users_h, pos_h, neg_h, rel_h, ent_h, relemb_h, outp_h, outn_h,
          uidx, pidx, nidx, ridx, bufs, op_v, on_v, sem0, sem1, sem2):
    wid = lax.axis_index("s") * _NC + lax.axis_index("c")
    base = wid * _BPW
    pltpu.sync_copy(users_h.at[pl.ds(base, _BPW)], uidx)
    pltpu.sync_copy(pos_h.at[pl.ds(base, _BPW)], pidx)
    pltpu.sync_copy(neg_h.at[pl.ds(base, _BPW)], nidx)
    pltpu.sync_copy(rel_h.at[pl.ds(base, _BPW)], ridx)

    sems = (sem0, sem1, sem2)
    lane = lax.iota(jnp.int32, _L)
    zero = jnp.zeros((_L,), jnp.float32)

    def fire(c, slot):
        off = c * _CH
        sem = sems[slot]
        u_b, r_b, p_b, n_b = (bufs.at[4 * slot + j] for j in range(4))

        def grp(g, _):
            s = pl.ds(off + g * _L, _L)
            uv = uidx[s]
            rv = ridx[s]
            pv = pidx[s]
            nv = nidx[s]
            for k in range(_L):
                d = pl.ds(g * _L + k, 1)
                ue, re_, pe, ne = uv[k], rv[k], pv[k], nv[k]
                pltpu.async_copy(
                    ent_h.at[lax.shift_right_logical(ue, 3)].at[pl.ds(ue & 7, 1)],
                    u_b.at[d], sem)
                pltpu.async_copy(
                    relemb_h.at[lax.shift_right_logical(re_, 3)].at[pl.ds(re_ & 7, 1)],
                    r_b.at[d], sem)
                pltpu.async_copy(
                    ent_h.at[lax.shift_right_logical(pe, 3)].at[pl.ds(pe & 7, 1)],
                    p_b.at[d], sem)
                pltpu.async_copy(
                    ent_h.at[lax.shift_right_logical(ne, 3)].at[pl.ds(ne & 7, 1)],
                    n_b.at[d], sem)
            return 0

        lax.fori_loop(0, _G, grp, 0)

    def drain(slot):
        sem = sems[slot]
        for j in range(4):
            pltpu.make_async_copy(ent_h.at[pl.ds(0, _CH // 8)],
                                  bufs.at[4 * slot + j].reshape(_CH // 8, 8, _D),
                                  sem).wait()

    fire(0, 0)
    if _NCH > 1:
        fire(1, 1)
    for c in range(_NCH):
        slot = c % 3
        if c + 2 < _NCH:
            fire(c + 2, (c + 2) % 3)
        drain(slot)
        off = c * _CH
        u_b, r_b, p_b, n_b = (bufs.at[4 * slot + j] for j in range(4))

        def group(g, _):
            sl = pl.ds(off + g * _L, _L)
            vp = zero
            vn = zero
            for k in range(_L):
                row = g * _L + k
                app = zero
                ann = zero
                for q in range(_D // _L):
                    cs = pl.ds(q * _L, _L)
                    t = u_b[row, cs] + r_b[row, cs]
                    dp = t - p_b[row, cs]
                    dn = t - n_b[row, cs]
                    app = app + dp * dp
                    ann = ann + dn * dn
                sp = jnp.sum(app)
                sn = jnp.sum(ann)
                vp = jnp.where(lane == k, sp, vp)
                vn = jnp.where(lane == k, sn, vn)
            op_v[sl] = _neg_sqrt(vp)
            on_v[sl] = _neg_sqrt(vn)
            return 0

        lax.fori_loop(0, _G, group, 0)

    pltpu.sync_copy(op_v, outp_h.at[pl.ds(base, _BPW)])
    pltpu.sync_copy(on_v, outn_h.at[pl.ds(base, _BPW)])


_sc_score = functools.partial(
    pl.kernel,
    out_type=(jax.ShapeDtypeStruct((_B,), jnp.float32),
              jax.ShapeDtypeStruct((_B,), jnp.float32)),
    mesh=plsc.VectorSubcoreMesh(core_axis_name="c", subcore_axis_name="s"),
    compiler_params=pltpu.CompilerParams(needs_layout_passes=False,
                                         disable_bounds_checks=True),
    scratch_types=[
        pltpu.VMEM((_BPW,), jnp.int32),
        pltpu.VMEM((_BPW,), jnp.int32),
        pltpu.VMEM((_BPW,), jnp.int32),
        pltpu.VMEM((_BPW,), jnp.int32),
        pltpu.VMEM((12, _CH, _D), jnp.float32),
        pltpu.VMEM((_BPW,), jnp.float32),
        pltpu.VMEM((_BPW,), jnp.float32),
        pltpu.SemaphoreType.DMA,
        pltpu.SemaphoreType.DMA,
        pltpu.SemaphoreType.DMA,
    ],
)(_body)


def kernel(users, pos_items, neg_items, relations, ent_emb, rel_emb):
    ent3 = ent_emb.reshape(ent_emb.shape[0] // 8, 8, _D)
    rel3 = rel_emb.reshape(rel_emb.shape[0] // 8, 8, _D)
    return _sc_score(users.astype(jnp.int32), pos_items.astype(jnp.int32),
                     neg_items.astype(jnp.int32), relations.astype(jnp.int32),
                     ent3, rel3)
